# Initial kernel scaffold; baseline (speedup 1.0000x reference)
import jax
import jax.numpy as jnp
from jax.experimental import pallas as pl


def kernel(x):
    raise NotImplementedError("write your pallas kernel here")


# speedup vs baseline: 1.4531x; 1.4531x over previous
import jax
import jax.numpy as jnp
from jax import lax
from jax.experimental import pallas as pl
from jax.experimental.pallas import tpu as pltpu
from jax.experimental.pallas import tpu_sc as plsc

NUM_KEEP = 4096

_NC = 2
_NS = 16
_NW = _NC * _NS

_CHUNK = 32


def _build_gather(rows_total: int, feat: int):
    rpw = rows_total // _NW
    nch = rpw // _CHUNK
    mesh = plsc.VectorSubcoreMesh(core_axis_name="c", subcore_axis_name="s")

    @pl.kernel(
        mesh=mesh,
        out_type=jax.ShapeDtypeStruct((rows_total, feat), jnp.float32),
        scratch_types=[
            pltpu.VMEM((rpw,), jnp.int32),
            pltpu.VMEM((_CHUNK, feat), jnp.float32),
            pltpu.VMEM((_CHUNK, feat), jnp.float32),
            pltpu.SemaphoreType.DMA,
            pltpu.SemaphoreType.DMA,
        ],
    )
    def gather_rows(table_hbm, idx_hbm, out_hbm, idx_v, buf0, buf1, sem0, sem1):
        wid = lax.axis_index("s") * _NC + lax.axis_index("c")
        base = wid * rpw
        pltpu.sync_copy(idx_hbm.at[pl.ds(base, rpw)], idx_v)
        bufs = (buf0, buf1)
        sems = (sem0, sem1)
        pending = [None, None]
        pending[0] = pltpu.async_copy(
            table_hbm.at[idx_v.at[pl.ds(0, _CHUNK)]], buf0, sem0)
        for c in range(nch):
            if c + 1 < nch:
                nxt = (c + 1) % 2
                pending[nxt] = pltpu.async_copy(
                    table_hbm.at[idx_v.at[pl.ds((c + 1) * _CHUNK, _CHUNK)]],
                    bufs[nxt], sems[nxt])
            cur = c % 2
            pending[cur].wait()
            pltpu.sync_copy(bufs[cur], out_hbm.at[pl.ds(base + c * _CHUNK, _CHUNK)])

    return gather_rows


def kernel(x):
    b, t, f = x.shape
    keep = min(t, NUM_KEEP)
    skey = jax.random.key(42)
    scores = jax.random.uniform(skey, (b, t), dtype=jnp.float32)
    idx = jnp.argsort(scores, axis=1)[:, :keep]
    flat_idx = (idx.astype(jnp.int32)
                + jnp.arange(b, dtype=jnp.int32)[:, None] * t).reshape(-1)
    table = x.reshape(b * t, f)
    out = _build_gather(b * keep, f)(table, flat_idx)
    return out.reshape(b, keep, f)
